# baseline (speedup 1.0000x reference)
import functools

import jax
import jax.numpy as jnp
from jax import lax
from jax.experimental import pallas as pl
from jax.experimental.pallas import tpu as pltpu
from jax.experimental.pallas import tpu_sc as plsc

_B = 512
_CH = 32


def _sc_info():
    info = plsc.get_sparse_core_info()
    return info.num_cores, info.num_subcores


def _make_expand_scatter(n_tok, n_cols, n_pad, kk, dtype):
    nc, ns = _sc_info()
    nw = nc * ns
    per_w = n_tok // nw
    nbuf = 3
    n_chunks = per_w // _CH
    mesh = plsc.VectorSubcoreMesh(core_axis_name="c", subcore_axis_name="s")

    @functools.partial(
        pl.kernel,
        mesh=mesh,
        out_type=jax.ShapeDtypeStruct((n_pad, n_cols), dtype),
        scratch_types=[
            pltpu.VMEM((kk, n_chunks, _CH), jnp.int32),
            pltpu.VMEM((_CH, n_cols), dtype),
            pltpu.VMEM((_CH, n_cols), dtype),
            pltpu.VMEM((_CH, n_cols), dtype),
        ] + [pltpu.SemaphoreType.DMA] * 6,
    )
    def expand(rows_hbm, pos_hbm, out_hbm, pos_v,
               b0, b1, b2, g0, g1, g2, s0, s1, s2):
        wid = lax.axis_index("s") * nc + lax.axis_index("c")
        base = wid * per_w
        bufs = (b0, b1, b2)
        gs = (g0, g1, g2)
        ss = (s0, s1, s2)
        pltpu.sync_copy(
            pos_hbm.at[:, pl.ds(wid * n_chunks, n_chunks), :], pos_v)

        def g_start(c):
            return pltpu.async_copy(
                rows_hbm.at[pl.ds(base + c * _CH, _CH)],
                bufs[c % nbuf], gs[c % nbuf])

        def s_start(c, j):
            return pltpu.async_copy(
                bufs[c % nbuf],
                out_hbm.at[pos_v.at[j, c]], ss[c % nbuf])

        gh = [None] * n_chunks
        sh = [[None, None] for _ in range(n_chunks)]
        for c in range(min(nbuf, n_chunks)):
            gh[c] = g_start(c)
        for c in range(n_chunks):
            gh[c].wait()
            for j in range(kk):
                sh[c][j] = s_start(c, j)
            nxt = c + nbuf
            if nxt < n_chunks:
                for j in range(kk):
                    sh[c][j].wait()
                gh[nxt] = g_start(nxt)
        for c in range(max(0, n_chunks - nbuf), n_chunks):
            for j in range(kk):
                sh[c][j].wait()

    return expand


def _make_row_gather(n_rows, n_cols, dtype):
    nc, ns = _sc_info()
    nw = nc * ns
    per_w = n_rows // nw
    nbuf = 3
    n_chunks = per_w // _CH
    mesh = plsc.VectorSubcoreMesh(core_axis_name="c", subcore_axis_name="s")

    @functools.partial(
        pl.kernel,
        mesh=mesh,
        out_type=jax.ShapeDtypeStruct((n_rows, n_cols), dtype),
        scratch_types=[
            pltpu.VMEM((per_w,), jnp.int32),
            pltpu.VMEM((_CH, n_cols), dtype),
            pltpu.VMEM((_CH, n_cols), dtype),
            pltpu.VMEM((_CH, n_cols), dtype),
        ] + [pltpu.SemaphoreType.DMA] * 6,
    )
    def gather_rows(table_hbm, idx_hbm, out_hbm, idx_v,
                    b0, b1, b2, g0, g1, g2, s0, s1, s2):
        wid = lax.axis_index("s") * nc + lax.axis_index("c")
        base = wid * per_w
        bufs = (b0, b1, b2)
        gs = (g0, g1, g2)
        ss = (s0, s1, s2)
        pltpu.sync_copy(idx_hbm.at[pl.ds(base, per_w)], idx_v)

        def g_start(c):
            return pltpu.async_copy(
                table_hbm.at[idx_v.at[pl.ds(c * _CH, _CH)]],
                bufs[c % nbuf], gs[c % nbuf])

        def s_start(c):
            return pltpu.async_copy(
                bufs[c % nbuf],
                out_hbm.at[pl.ds(base + c * _CH, _CH)], ss[c % nbuf])

        gh = [None] * n_chunks
        sh = [None] * n_chunks
        for c in range(min(nbuf, n_chunks)):
            gh[c] = g_start(c)
        for c in range(n_chunks):
            gh[c].wait()
            sh[c] = s_start(c)
            nxt = c + nbuf
            if nxt < n_chunks:
                sh[c].wait()
                gh[nxt] = g_start(nxt)
        for c in range(max(0, n_chunks - nbuf), n_chunks):
            sh[c].wait()

    return gather_rows


def _ffn_body(f, blk_e_ref, blk_row_ref, valid_ref, xs_ref, guw_ref, dw_ref,
              ys_ref):
    i = pl.program_id(0)

    @pl.when(valid_ref[i] == 1)
    def _compute():
        fb = 128
        x = xs_ref[...]
        for jf in range(f // fb):
            lo = jf * fb
            gw = guw_ref[0, lo:lo + fb, :]
            uw = guw_ref[0, f + lo:f + lo + fb, :]
            g = lax.dot_general(x, gw, (((1,), (1,)), ((), ())),
                                preferred_element_type=jnp.float32)
            u = lax.dot_general(x, uw, (((1,), (1,)), ((), ())),
                                preferred_element_type=jnp.float32)
            inter = g * lax.logistic(g) * u
            dwj = dw_ref[0, :, lo:lo + fb]
            part = lax.dot_general(inter, dwj, (((1,), (1,)), ((), ())),
                                   preferred_element_type=jnp.float32)
            if jf == 0:
                ys_ref[...] = part
            else:
                ys_ref[...] = ys_ref[...] + part


def _pairsum_body(h, y0_ref, y1_ref, w_ref, o_ref):
    w = w_ref[...]
    o_ref[...] = y0_ref[...] * w[:, 0:1] + y1_ref[...] * w[:, 1:2]


def kernel(hidden_states, topk_weights, topk_ids, gate_up_weights, down_weights):
    t, h = hidden_states.shape
    k = topk_ids.shape[1]
    e = gate_up_weights.shape[0]
    f = down_weights.shape[2]
    s = t * k
    nb = s // _B + e
    p = nb * _B

    flat_ids = topk_ids.reshape(-1).astype(jnp.int32)
    onehot = (flat_ids[:, None] == jnp.arange(e, dtype=jnp.int32)[None, :])
    ranks_all = jnp.cumsum(onehot.astype(jnp.int32), axis=0)
    counts = ranks_all[-1]
    rank = jnp.sum(jnp.where(onehot, ranks_all - 1, 0), axis=1)
    blocks_per_e = (counts + _B - 1) // _B
    cumb = jnp.cumsum(blocks_per_e).astype(jnp.int32)
    nused = cumb[-1]
    expert_pad_start = ((cumb - blocks_per_e) * _B).astype(jnp.int32)

    bi = jnp.arange(nb, dtype=jnp.int32)
    blk_e_raw = jnp.sum((cumb[None, :] <= bi[:, None]).astype(jnp.int32),
                        axis=1)
    last_e = jnp.sum((cumb <= nused - 1).astype(jnp.int32))
    valid_blk = bi < nused
    blk_e = jnp.where(valid_blk, jnp.minimum(blk_e_raw, e - 1), last_e)
    blk_row = jnp.where(valid_blk, bi, nused - 1).astype(jnp.int32)
    blk_valid = valid_blk.astype(jnp.int32)

    pos_of_slot = (jnp.sum(jnp.where(onehot, expert_pad_start[None, :], 0),
                           axis=1) + rank).astype(jnp.int32)
    pos_kt = pos_of_slot.reshape(t, k).T.reshape(k, t // _CH, _CH)

    xs = _make_expand_scatter(t, h, p, k, jnp.float32)(hidden_states, pos_kt)

    grid_spec = pltpu.PrefetchScalarGridSpec(
        num_scalar_prefetch=3,
        grid=(nb,),
        in_specs=[
            pl.BlockSpec((_B, h), lambda i, be, br, vv: (br[i], 0)),
            pl.BlockSpec((1, 2 * f, h), lambda i, be, br, vv: (be[i], 0, 0)),
            pl.BlockSpec((1, h, f), lambda i, be, br, vv: (be[i], 0, 0)),
        ],
        out_specs=pl.BlockSpec((_B, h), lambda i, be, br, vv: (br[i], 0)),
    )
    ys = pl.pallas_call(
        functools.partial(_ffn_body, f),
        grid_spec=grid_spec,
        out_shape=jax.ShapeDtypeStruct((p, h), jnp.float32),
        compiler_params=pltpu.CompilerParams(
            dimension_semantics=("arbitrary",),
            vmem_limit_bytes=100 * 1024 * 1024),
    )(blk_e, blk_row, blk_valid, xs, gate_up_weights, down_weights)

    ys_slab = _make_row_gather(s, h, jnp.float32)(ys, pos_kt.reshape(s))

    bt = 1024
    nt = t // bt
    out = pl.pallas_call(
        functools.partial(_pairsum_body, h),
        grid=(nt,),
        in_specs=[
            pl.BlockSpec((bt, h), lambda i: (i, 0)),
            pl.BlockSpec((bt, h), lambda i: (i + nt, 0)),
            pl.BlockSpec((bt, k), lambda i: (i, 0)),
        ],
        out_specs=pl.BlockSpec((bt, h), lambda i: (i, 0)),
        out_shape=jax.ShapeDtypeStruct((t, h), jnp.float32),
    )(ys_slab, ys_slab, topk_weights)
    return out


# speedup vs baseline: 1.3919x; 1.3919x over previous
import functools

import jax
import jax.numpy as jnp
from jax import lax
from jax.experimental import pallas as pl
from jax.experimental.pallas import tpu as pltpu
from jax.experimental.pallas import tpu_sc as plsc

_B = 512
_CH = 32


def _sc_info():
    info = plsc.get_sparse_core_info()
    return info.num_cores, info.num_subcores


def _make_expand_scatter(n_tok, n_cols, n_pad, kk, dtype):
    nc, ns = _sc_info()
    nw = nc * ns
    per_w = n_tok // nw
    nbuf = 3
    n_chunks = per_w // _CH
    mesh = plsc.VectorSubcoreMesh(core_axis_name="c", subcore_axis_name="s")

    @functools.partial(
        pl.kernel,
        mesh=mesh,
        out_type=jax.ShapeDtypeStruct((n_pad, n_cols), dtype),
        scratch_types=[
            pltpu.VMEM((kk, n_chunks, _CH), jnp.int32),
            pltpu.VMEM((_CH, n_cols), dtype),
            pltpu.VMEM((_CH, n_cols), dtype),
            pltpu.VMEM((_CH, n_cols), dtype),
        ] + [pltpu.SemaphoreType.DMA] * 6,
    )
    def expand(rows_hbm, pos_hbm, out_hbm, pos_v,
               b0, b1, b2, g0, g1, g2, s0, s1, s2):
        wid = lax.axis_index("s") * nc + lax.axis_index("c")
        base = wid * per_w
        bufs = (b0, b1, b2)
        gs = (g0, g1, g2)
        ss = (s0, s1, s2)
        pltpu.sync_copy(
            pos_hbm.at[:, pl.ds(wid * n_chunks, n_chunks), :], pos_v)

        def g_start(c):
            return pltpu.async_copy(
                rows_hbm.at[pl.ds(base + c * _CH, _CH)],
                bufs[c % nbuf], gs[c % nbuf])

        def s_start(c, j):
            return pltpu.async_copy(
                bufs[c % nbuf],
                out_hbm.at[pos_v.at[j, c]], ss[c % nbuf])

        gh = [None] * n_chunks
        sh = [[None, None] for _ in range(n_chunks)]
        for c in range(min(nbuf, n_chunks)):
            gh[c] = g_start(c)
        for c in range(n_chunks):
            gh[c].wait()
            for j in range(kk):
                sh[c][j] = s_start(c, j)
            nxt = c + nbuf
            if nxt < n_chunks:
                for j in range(kk):
                    sh[c][j].wait()
                gh[nxt] = g_start(nxt)
        for c in range(max(0, n_chunks - nbuf), n_chunks):
            for j in range(kk):
                sh[c][j].wait()

    return expand


def _make_row_gather(n_rows, n_cols, dtype):
    nc, ns = _sc_info()
    nw = nc * ns
    per_w = n_rows // nw
    nbuf = 3
    n_chunks = per_w // _CH
    mesh = plsc.VectorSubcoreMesh(core_axis_name="c", subcore_axis_name="s")

    @functools.partial(
        pl.kernel,
        mesh=mesh,
        out_type=jax.ShapeDtypeStruct((n_rows, n_cols), dtype),
        scratch_types=[
            pltpu.VMEM((per_w,), jnp.int32),
            pltpu.VMEM((_CH, n_cols), dtype),
            pltpu.VMEM((_CH, n_cols), dtype),
            pltpu.VMEM((_CH, n_cols), dtype),
        ] + [pltpu.SemaphoreType.DMA] * 6,
    )
    def gather_rows(table_hbm, idx_hbm, out_hbm, idx_v,
                    b0, b1, b2, g0, g1, g2, s0, s1, s2):
        wid = lax.axis_index("s") * nc + lax.axis_index("c")
        base = wid * per_w
        bufs = (b0, b1, b2)
        gs = (g0, g1, g2)
        ss = (s0, s1, s2)
        pltpu.sync_copy(idx_hbm.at[pl.ds(base, per_w)], idx_v)

        def g_start(c):
            return pltpu.async_copy(
                table_hbm.at[idx_v.at[pl.ds(c * _CH, _CH)]],
                bufs[c % nbuf], gs[c % nbuf])

        def s_start(c):
            return pltpu.async_copy(
                bufs[c % nbuf],
                out_hbm.at[pl.ds(base + c * _CH, _CH)], ss[c % nbuf])

        gh = [None] * n_chunks
        sh = [None] * n_chunks
        for c in range(min(nbuf, n_chunks)):
            gh[c] = g_start(c)
        for c in range(n_chunks):
            gh[c].wait()
            sh[c] = s_start(c)
            nxt = c + nbuf
            if nxt < n_chunks:
                sh[c].wait()
                gh[nxt] = g_start(nxt)
        for c in range(max(0, n_chunks - nbuf), n_chunks):
            sh[c].wait()

    return gather_rows


def _ffn_body(f, blk_e_ref, blk_row_ref, valid_ref, xs_ref, guw_ref, dw_ref,
              ys_ref):
    i = pl.program_id(0)

    @pl.when(valid_ref[i] == 1)
    def _compute():
        fb = 256
        x = xs_ref[...]
        for jf in range(f // fb):
            lo = jf * fb
            gw = guw_ref[0, lo:lo + fb, :]
            uw = guw_ref[0, f + lo:f + lo + fb, :]
            g = lax.dot_general(x, gw, (((1,), (1,)), ((), ())),
                                preferred_element_type=jnp.float32)
            u = lax.dot_general(x, uw, (((1,), (1,)), ((), ())),
                                preferred_element_type=jnp.float32)
            inter = g * lax.logistic(g) * u
            dwj = dw_ref[0, :, lo:lo + fb]
            part = lax.dot_general(inter, dwj, (((1,), (1,)), ((), ())),
                                   preferred_element_type=jnp.float32)
            if jf == 0:
                ys_ref[...] = part
            else:
                ys_ref[...] = ys_ref[...] + part


def _pairsum_body(h, y0_ref, y1_ref, w_ref, o_ref):
    w = w_ref[...]
    o_ref[...] = y0_ref[...] * w[:, 0:1] + y1_ref[...] * w[:, 1:2]


def kernel(hidden_states, topk_weights, topk_ids, gate_up_weights, down_weights):
    t, h = hidden_states.shape
    k = topk_ids.shape[1]
    e = gate_up_weights.shape[0]
    f = down_weights.shape[2]
    s = t * k
    nb = s // _B + e
    p = nb * _B

    flat_ids = topk_ids.reshape(-1).astype(jnp.int32)
    onehot = (flat_ids[:, None] == jnp.arange(e, dtype=jnp.int32)[None, :])
    ranks_all = jnp.cumsum(onehot.astype(jnp.int32), axis=0)
    counts = ranks_all[-1]
    rank = jnp.sum(jnp.where(onehot, ranks_all - 1, 0), axis=1)
    blocks_per_e = (counts + _B - 1) // _B
    cumb = jnp.cumsum(blocks_per_e).astype(jnp.int32)
    nused = cumb[-1]
    expert_pad_start = ((cumb - blocks_per_e) * _B).astype(jnp.int32)

    bi = jnp.arange(nb, dtype=jnp.int32)
    blk_e_raw = jnp.sum((cumb[None, :] <= bi[:, None]).astype(jnp.int32),
                        axis=1)
    last_e = jnp.sum((cumb <= nused - 1).astype(jnp.int32))
    valid_blk = bi < nused
    blk_e = jnp.where(valid_blk, jnp.minimum(blk_e_raw, e - 1), last_e)
    blk_row = jnp.where(valid_blk, bi, nused - 1).astype(jnp.int32)
    blk_valid = valid_blk.astype(jnp.int32)

    pos_of_slot = (jnp.sum(jnp.where(onehot, expert_pad_start[None, :], 0),
                           axis=1) + rank).astype(jnp.int32)
    pos_kt = pos_of_slot.reshape(t, k).T.reshape(k, t // _CH, _CH)

    xs = _make_expand_scatter(t, h, p, k, jnp.float32)(hidden_states, pos_kt)

    grid_spec = pltpu.PrefetchScalarGridSpec(
        num_scalar_prefetch=3,
        grid=(nb,),
        in_specs=[
            pl.BlockSpec((_B, h), lambda i, be, br, vv: (br[i], 0)),
            pl.BlockSpec((1, 2 * f, h), lambda i, be, br, vv: (be[i], 0, 0)),
            pl.BlockSpec((1, h, f), lambda i, be, br, vv: (be[i], 0, 0)),
        ],
        out_specs=pl.BlockSpec((_B, h), lambda i, be, br, vv: (br[i], 0)),
    )
    ys = pl.pallas_call(
        functools.partial(_ffn_body, f),
        grid_spec=grid_spec,
        out_shape=jax.ShapeDtypeStruct((p, h), jnp.float32),
        compiler_params=pltpu.CompilerParams(
            dimension_semantics=("arbitrary",),
            vmem_limit_bytes=100 * 1024 * 1024),
    )(blk_e, blk_row, blk_valid, xs, gate_up_weights, down_weights)

    ys_slab = _make_row_gather(s, h, jnp.float32)(ys, pos_kt.reshape(s))

    bt = 1024
    nt = t // bt
    out = pl.pallas_call(
        functools.partial(_pairsum_body, h),
        grid=(nt,),
        in_specs=[
            pl.BlockSpec((bt, h), lambda i: (i, 0)),
            pl.BlockSpec((bt, h), lambda i: (i + nt, 0)),
            pl.BlockSpec((bt, k), lambda i: (i, 0)),
        ],
        out_specs=pl.BlockSpec((bt, h), lambda i: (i, 0)),
        out_shape=jax.ShapeDtypeStruct((t, h), jnp.float32),
    )(ys_slab, ys_slab, topk_weights)
    return out
